# baseline (speedup 1.0000x reference)
import functools

import jax
import jax.numpy as jnp
import numpy as np
from jax import lax
from jax.experimental import pallas as pl
from jax.experimental.pallas import tpu as pltpu
from jax.experimental.pallas import tpu_sc as plsc

_B = 128
_S = 32768
_K = 64
_L = 16
_NC = 2
_NS = 16
_NW = _NC * _NS
_RPW = _B // _NW
_NVEC = _S // _L
_NGRP = _NVEC // 16
_CAP = 512
_CVEC = _CAP // _L
_NBUF = 3

_MIN32 = np.int32(-2147483648)
_TOPBIT = np.uint32(0x80000000)


def _ukey16(v16):
    i = lax.bitcast_convert_type(v16, jnp.int32)
    mm = (i >> 31) | _MIN32
    return lax.bitcast_convert_type(i ^ mm, jnp.uint32)


def _unkey16(k16):
    si = lax.bitcast_convert_type(k16, jnp.int32)
    mm = (~(si >> 31)) | _MIN32
    return lax.bitcast_convert_type(si ^ mm, jnp.float32)


def _treemax(vals):
    while len(vals) > 1:
        vals = [jnp.maximum(vals[2 * i], vals[2 * i + 1])
                for i in range(len(vals) // 2)] + vals[len(vals) & ~1:]
    return vals[0]


def _bisect_key(count_fn):
    def step(_, carry):
        thr, bit = carry
        cand = thr | bit
        cnt = count_fn(cand)
        return (jnp.where(cnt >= _K, cand, thr), bit >> jnp.uint32(1))

    thr0 = jnp.zeros((_L,), jnp.uint32)
    bit0 = jnp.full((_L,), _TOPBIT, jnp.uint32)
    thr, _ = lax.fori_loop(0, 32, step, (thr0, bit0))
    return thr


def _row_select(xb, m1v, cv):
    @plsc.parallel_loop(0, _NGRP, unroll=2)
    def _a(g):
        vs = [xb[pl.ds((g * 16 + t) * _L, _L)] for t in range(16)]
        m1v[pl.ds(g * _L, _L)] = _treemax(vs)

    m2k = []
    for z in range(8):
        vs = [m1v[pl.ds((z * 16 + t) * _L, _L)] for t in range(16)]
        m2k.append(_ukey16(_treemax(vs)))

    def count_m2(cand):
        cnt = jnp.zeros((_L,), jnp.int32)
        for mk in m2k:
            cnt = cnt + plsc.all_reduce_population_count(mk >= cand)
        return cnt

    vf = _unkey16(_bisect_key(count_m2))

    ninf16 = jnp.full((_L,), -jnp.inf, jnp.float32)
    for q in range(_CVEC):
        cv[pl.ds(q * _L, _L)] = ninf16
    lane = lax.iota(jnp.int32, _L)
    half = _CAP // 2

    @plsc.parallel_loop(0, _NVEC // 2, unroll=8,
                        carry=(jnp.zeros((_L,), jnp.int32),
                               jnp.full((_L,), half, jnp.int32)))
    def cnts(j, carry):
        ca, cb = carry
        xa = xb[pl.ds((2 * j) * _L, _L)]
        xc = xb[pl.ds((2 * j + 1) * _L, _L)]
        ma = xa >= vf
        mb = xc >= vf
        oka = jnp.logical_and(ma, ca < half)
        okb = jnp.logical_and(mb, cb < _CAP)
        plsc.store_scatter(cv, [ca + lane], xa, mask=oka)
        plsc.store_scatter(cv, [cb + lane], xc, mask=okb)
        return (ca + jnp.where(ma, _L, 0), cb + jnp.where(mb, _L, 0))

    ca_fin, cb_fin = cnts
    nsa = jnp.max(ca_fin) >> 4
    nsb = (jnp.max(cb_fin) - half) >> 4
    nslot = jnp.maximum(nsa, nsb)
    hvec = _CVEC // 2

    def count_cand(cand):
        def cnt(q, acc):
            cka = _ukey16(cv[pl.ds(q * _L, _L)])
            ckb = _ukey16(cv[pl.ds((hvec + q) * _L, _L)])
            acc = acc + plsc.all_reduce_population_count(cka >= cand)
            return acc + plsc.all_reduce_population_count(ckb >= cand)

        return lax.fori_loop(0, jnp.minimum(nslot, hvec), cnt,
                             jnp.zeros((_L,), jnp.int32))

    tkey_cand = _bisect_key(count_cand)

    def full_bisect(_):
        def count_full(cand):
            def cnt(j, acc):
                ku = _ukey16(xb[pl.ds(j * _L, _L)])
                return acc + plsc.all_reduce_population_count(ku >= cand)

            return lax.fori_loop(0, _NVEC, cnt, jnp.zeros((_L,), jnp.int32))

        return _bisect_key(count_full)

    tkey = lax.cond(nslot > hvec, full_bisect, lambda _: tkey_cand, 0)
    return _unkey16(tkey)


def _sc_body(x_hbm, out_hbm, xball, m1v, cv, sin, sout):
    wid = lax.axis_index("s") * _NC + lax.axis_index("c")
    base = wid * _RPW * _S

    def in_copy(r):
        b = r % _NBUF
        return pltpu.async_copy(
            x_hbm.at[pl.ds(base + r * _S, _S)],
            xball.at[pl.ds(b * _S, _S)], sin.at[b])

    def out_copy(r):
        b = r % _NBUF
        return pltpu.async_copy(
            xball.at[pl.ds(b * _S, _S)],
            out_hbm.at[pl.ds(base + r * _S, _S)], sout.at[b])

    in_copy(0)
    in_copy(1)

    def row_body(r, _carry):
        b = r % _NBUF
        pltpu.make_async_copy(
            x_hbm.at[pl.ds(base + r * _S, _S)],
            xball.at[pl.ds(b * _S, _S)], sin.at[b]).wait()
        xb = xball.at[pl.ds(b * _S, _S)]
        tf = _row_select(xb, m1v, cv)

        @plsc.parallel_loop(0, _NVEC // 4, unroll=4)
        def _e(j):
            for u in range(4):
                x16 = xb[pl.ds((j * 4 + u) * _L, _L)]
                xb[pl.ds((j * 4 + u) * _L, _L)] = jnp.where(
                    x16 >= tf, jnp.float32(1.0), jnp.float32(0.0))

        out_copy(r)

        @pl.when(jnp.logical_and(r + _NBUF - 1 < _RPW, r >= 1))
        def _wait_prev_out():
            b2 = (r + _NBUF - 1) % _NBUF
            pltpu.make_async_copy(
                xball.at[pl.ds(b2 * _S, _S)],
                out_hbm.at[pl.ds(0, _S)], sout.at[b2]).wait()

        @pl.when(r + _NBUF - 1 < _RPW)
        def _start_next_in():
            in_copy(r + _NBUF - 1)

        return _carry

    lax.fori_loop(0, _RPW, row_body, 0)

    for r in range(_RPW - _NBUF, _RPW):
        b = r % _NBUF
        pltpu.make_async_copy(
            xball.at[pl.ds(b * _S, _S)],
            out_hbm.at[pl.ds(base + r * _S, _S)], sout.at[b]).wait()


_mesh = plsc.VectorSubcoreMesh(core_axis_name="c", subcore_axis_name="s")

_sc_call = functools.partial(
    pl.kernel,
    out_type=jax.ShapeDtypeStruct((_B * _S,), jnp.float32),
    mesh=_mesh,
    scratch_types=[
        pltpu.VMEM((_NBUF * _S,), jnp.float32),
        pltpu.VMEM((_NGRP * _L,), jnp.float32),
        pltpu.VMEM((_CAP,), jnp.float32),
        pltpu.SemaphoreType.DMA((_NBUF,)),
        pltpu.SemaphoreType.DMA((_NBUF,)),
    ],
    compiler_params=pltpu.CompilerParams(needs_layout_passes=False),
)(_sc_body)


@jax.jit
def kernel(logits):
    x = logits.reshape(_B * _S)
    return _sc_call(x).reshape(_B, _S, 1)


# speedup vs baseline: 1.1586x; 1.1586x over previous
import functools

import jax
import jax.numpy as jnp
import numpy as np
from jax import lax
from jax.experimental import pallas as pl
from jax.experimental.pallas import tpu as pltpu
from jax.experimental.pallas import tpu_sc as plsc

_B = 128
_S = 32768
_K = 64
_L = 16
_NC = 2
_NS = 16
_NW = _NC * _NS
_RPW = _B // _NW
_NVEC = _S // _L
_NGRP = _NVEC // 16
_CAP = 512
_CVEC = _CAP // _L
_NBUF = 3

_MIN32 = np.int32(-2147483648)
_TOPBIT = np.uint32(0x80000000)


def _ukey16(v16):
    i = lax.bitcast_convert_type(v16, jnp.int32)
    mm = (i >> 31) | _MIN32
    return lax.bitcast_convert_type(i ^ mm, jnp.uint32)


def _unkey16(k16):
    si = lax.bitcast_convert_type(k16, jnp.int32)
    mm = (~(si >> 31)) | _MIN32
    return lax.bitcast_convert_type(si ^ mm, jnp.float32)


def _treemax(vals):
    while len(vals) > 1:
        vals = [jnp.maximum(vals[2 * i], vals[2 * i + 1])
                for i in range(len(vals) // 2)] + vals[len(vals) & ~1:]
    return vals[0]


def _bisect_key(count_fn):
    def step(_, carry):
        thr, bit = carry
        cand = thr | bit
        cnt = count_fn(cand)
        return (jnp.where(cnt >= _K, cand, thr), bit >> jnp.uint32(1))

    thr0 = jnp.zeros((_L,), jnp.uint32)
    bit0 = jnp.full((_L,), _TOPBIT, jnp.uint32)
    thr, _ = lax.fori_loop(0, 32, step, (thr0, bit0))
    return thr


def _row_select(xb, m1v, cv):
    @plsc.parallel_loop(0, _NGRP, unroll=2)
    def _a(g):
        vs = [xb[pl.ds((g * 16 + t) * _L, _L)] for t in range(16)]
        m1v[pl.ds(g * _L, _L)] = _treemax(vs)

    m2k = []
    for z in range(8):
        vs = [m1v[pl.ds((z * 16 + t) * _L, _L)] for t in range(16)]
        m2k.append(_ukey16(_treemax(vs)))

    def count_m2(cand):
        cnt = jnp.zeros((_L,), jnp.int32)
        for mk in m2k:
            cnt = cnt + plsc.all_reduce_population_count(mk >= cand)
        return cnt

    vf = _unkey16(_bisect_key(count_m2))

    ninf16 = jnp.full((_L,), -jnp.inf, jnp.float32)
    for q in range(_CVEC):
        cv[pl.ds(q * _L, _L)] = ninf16
    lane = lax.iota(jnp.int32, _L)
    half = _CAP // 2

    @plsc.parallel_loop(0, _NVEC // 2, unroll=4,
                        carry=(jnp.zeros((_L,), jnp.int32),
                               jnp.full((_L,), half, jnp.int32)))
    def cnts(j, carry):
        ca, cb = carry
        xa = xb[pl.ds((2 * j) * _L, _L)]
        xc = xb[pl.ds((2 * j + 1) * _L, _L)]
        ma = xa >= vf
        mb = xc >= vf
        oka = jnp.logical_and(ma, ca < half)
        okb = jnp.logical_and(mb, cb < _CAP)
        plsc.store_scatter(cv, [ca + lane], xa, mask=oka)
        plsc.store_scatter(cv, [cb + lane], xc, mask=okb)
        return (ca + jnp.where(ma, _L, 0), cb + jnp.where(mb, _L, 0))

    ca_fin, cb_fin = cnts
    nsa = jnp.max(ca_fin) >> 4
    nsb = (jnp.max(cb_fin) - half) >> 4
    nslot = jnp.maximum(nsa, nsb)
    hvec = _CVEC // 2

    def count_cand(cand):
        def cnt(q, acc):
            cka = _ukey16(cv[pl.ds(q * _L, _L)])
            ckb = _ukey16(cv[pl.ds((hvec + q) * _L, _L)])
            acc = acc + plsc.all_reduce_population_count(cka >= cand)
            return acc + plsc.all_reduce_population_count(ckb >= cand)

        return lax.fori_loop(0, jnp.minimum(nslot, hvec), cnt,
                             jnp.zeros((_L,), jnp.int32))

    tkey_cand = _bisect_key(count_cand)

    def full_bisect(_):
        def count_full(cand):
            def cnt(j, acc):
                ku = _ukey16(xb[pl.ds(j * _L, _L)])
                return acc + plsc.all_reduce_population_count(ku >= cand)

            return lax.fori_loop(0, _NVEC, cnt, jnp.zeros((_L,), jnp.int32))

        return _bisect_key(count_full)

    tkey = lax.cond(nslot > hvec, full_bisect, lambda _: tkey_cand, 0)
    return _unkey16(tkey)


def _sc_body(x_hbm, out_hbm, xball, m1v, cv, sin, sout):
    wid = lax.axis_index("s") * _NC + lax.axis_index("c")
    base = wid * _RPW * _S

    def in_copy(r):
        b = r % _NBUF
        return pltpu.async_copy(
            x_hbm.at[pl.ds(base + r * _S, _S)],
            xball.at[pl.ds(b * _S, _S)], sin.at[b])

    def out_copy(r):
        b = r % _NBUF
        return pltpu.async_copy(
            xball.at[pl.ds(b * _S, _S)],
            out_hbm.at[pl.ds(base + r * _S, _S)], sout.at[b])

    in_copy(0)
    in_copy(1)

    def row_body(r, _carry):
        b = r % _NBUF
        pltpu.make_async_copy(
            x_hbm.at[pl.ds(base + r * _S, _S)],
            xball.at[pl.ds(b * _S, _S)], sin.at[b]).wait()
        xb = xball.at[pl.ds(b * _S, _S)]
        tf = _row_select(xb, m1v, cv)

        @plsc.parallel_loop(0, _NVEC // 4, unroll=2)
        def _e(j):
            for u in range(4):
                x16 = xb[pl.ds((j * 4 + u) * _L, _L)]
                xb[pl.ds((j * 4 + u) * _L, _L)] = jnp.where(
                    x16 >= tf, jnp.float32(1.0), jnp.float32(0.0))

        out_copy(r)

        @pl.when(jnp.logical_and(r + _NBUF - 1 < _RPW, r >= 1))
        def _wait_prev_out():
            b2 = (r + _NBUF - 1) % _NBUF
            pltpu.make_async_copy(
                xball.at[pl.ds(b2 * _S, _S)],
                out_hbm.at[pl.ds(0, _S)], sout.at[b2]).wait()

        @pl.when(r + _NBUF - 1 < _RPW)
        def _start_next_in():
            in_copy(r + _NBUF - 1)

        return _carry

    lax.fori_loop(0, _RPW, row_body, 0)

    for r in range(_RPW - _NBUF, _RPW):
        b = r % _NBUF
        pltpu.make_async_copy(
            xball.at[pl.ds(b * _S, _S)],
            out_hbm.at[pl.ds(base + r * _S, _S)], sout.at[b]).wait()


_mesh = plsc.VectorSubcoreMesh(core_axis_name="c", subcore_axis_name="s")

_sc_call = functools.partial(
    pl.kernel,
    out_type=jax.ShapeDtypeStruct((_B * _S,), jnp.float32),
    mesh=_mesh,
    scratch_types=[
        pltpu.VMEM((_NBUF * _S,), jnp.float32),
        pltpu.VMEM((_NGRP * _L,), jnp.float32),
        pltpu.VMEM((_CAP,), jnp.float32),
        pltpu.SemaphoreType.DMA((_NBUF,)),
        pltpu.SemaphoreType.DMA((_NBUF,)),
    ],
    compiler_params=pltpu.CompilerParams(needs_layout_passes=False),
)(_sc_body)


@jax.jit
def kernel(logits):
    x = logits.reshape(_B * _S)
    return _sc_call(x).reshape(_B, _S, 1)


# speedup vs baseline: 1.2661x; 1.0927x over previous
import functools

import jax
import jax.numpy as jnp
import numpy as np
from jax import lax
from jax.experimental import pallas as pl
from jax.experimental.pallas import tpu as pltpu
from jax.experimental.pallas import tpu_sc as plsc

_B = 128
_S = 32768
_K = 64
_L = 16
_NC = 2
_NS = 16
_NW = _NC * _NS
_RPW = _B // _NW
_NVEC = _S // _L
_NGRP = _NVEC // 16
_CAP = 512
_CVEC = _CAP // _L
_NBUF = 3

_MIN32 = np.int32(-2147483648)
_TOPBIT = np.uint32(0x80000000)


def _ukey16(v16):
    i = lax.bitcast_convert_type(v16, jnp.int32)
    mm = (i >> 31) | _MIN32
    return lax.bitcast_convert_type(i ^ mm, jnp.uint32)


def _unkey16(k16):
    si = lax.bitcast_convert_type(k16, jnp.int32)
    mm = (~(si >> 31)) | _MIN32
    return lax.bitcast_convert_type(si ^ mm, jnp.float32)


def _treemax(vals):
    while len(vals) > 1:
        vals = [jnp.maximum(vals[2 * i], vals[2 * i + 1])
                for i in range(len(vals) // 2)] + vals[len(vals) & ~1:]
    return vals[0]


def _bisect_key(count_fn):
    def step(_, carry):
        thr, bit = carry
        cand = thr | bit
        cnt = count_fn(cand)
        return (jnp.where(cnt >= _K, cand, thr), bit >> jnp.uint32(1))

    thr0 = jnp.zeros((_L,), jnp.uint32)
    bit0 = jnp.full((_L,), _TOPBIT, jnp.uint32)
    thr, _ = lax.fori_loop(0, 32, step, (thr0, bit0))
    return thr


def _row_select(xb, cv):
    ninf = jnp.full((_L,), -jnp.inf, jnp.float32)
    seg = _NVEC // 8

    @plsc.parallel_loop(0, seg, unroll=2, carry=(ninf,) * 8)
    def accs(j, acc):
        return tuple(
            jnp.maximum(acc[z], xb[pl.ds((z * seg + j) * _L, _L)])
            for z in range(8))

    m2k = [_ukey16(a) for a in accs]

    def count_m2(cand):
        cnt = jnp.zeros((_L,), jnp.int32)
        for mk in m2k:
            cnt = cnt + plsc.all_reduce_population_count(mk >= cand)
        return cnt

    vf = _unkey16(_bisect_key(count_m2))

    ninf16 = jnp.full((_L,), -jnp.inf, jnp.float32)
    for q in range(_CVEC):
        cv[pl.ds(q * _L, _L)] = ninf16
    lane = lax.iota(jnp.int32, _L)
    half = _CAP // 2

    @plsc.parallel_loop(0, _NVEC // 2, unroll=4,
                        carry=(jnp.zeros((_L,), jnp.int32),
                               jnp.full((_L,), half, jnp.int32)))
    def cnts(j, carry):
        ca, cb = carry
        xa = xb[pl.ds((2 * j) * _L, _L)]
        xc = xb[pl.ds((2 * j + 1) * _L, _L)]
        ma = xa >= vf
        mb = xc >= vf
        oka = jnp.logical_and(ma, ca < half)
        okb = jnp.logical_and(mb, cb < _CAP)
        plsc.store_scatter(cv, [ca + lane], xa, mask=oka)
        plsc.store_scatter(cv, [cb + lane], xc, mask=okb)
        return (ca + jnp.where(ma, _L, 0), cb + jnp.where(mb, _L, 0))

    ca_fin, cb_fin = cnts
    nsa = jnp.max(ca_fin) >> 4
    nsb = (jnp.max(cb_fin) - half) >> 4
    nslot = jnp.maximum(nsa, nsb)
    hvec = _CVEC // 2

    def count_cand(cand):
        acc = jnp.zeros((_L,), jnp.int32)
        for q in range(8):
            cka = _ukey16(cv[pl.ds(q * _L, _L)])
            ckb = _ukey16(cv[pl.ds((hvec + q) * _L, _L)])
            acc = acc + plsc.all_reduce_population_count(cka >= cand)
            acc = acc + plsc.all_reduce_population_count(ckb >= cand)

        def cnt(q, a):
            cka = _ukey16(cv[pl.ds(q * _L, _L)])
            ckb = _ukey16(cv[pl.ds((hvec + q) * _L, _L)])
            a = a + plsc.all_reduce_population_count(cka >= cand)
            return a + plsc.all_reduce_population_count(ckb >= cand)

        return lax.fori_loop(8, jnp.clip(nslot, 8, hvec), cnt, acc)

    tkey_cand = _bisect_key(count_cand)

    def full_bisect(_):
        def count_full(cand):
            def cnt(j, acc):
                ku = _ukey16(xb[pl.ds(j * _L, _L)])
                return acc + plsc.all_reduce_population_count(ku >= cand)

            return lax.fori_loop(0, _NVEC, cnt, jnp.zeros((_L,), jnp.int32))

        return _bisect_key(count_full)

    tkey = lax.cond(nslot > hvec, full_bisect, lambda _: tkey_cand, 0)
    return _unkey16(tkey)


def _sc_body(x_hbm, out_hbm, xball, cv, sin, sout):
    wid = lax.axis_index("s") * _NC + lax.axis_index("c")
    base = wid * _RPW * _S

    def in_copy(r):
        b = r % _NBUF
        return pltpu.async_copy(
            x_hbm.at[pl.ds(base + r * _S, _S)],
            xball.at[pl.ds(b * _S, _S)], sin.at[b])

    def out_copy(r):
        b = r % _NBUF
        return pltpu.async_copy(
            xball.at[pl.ds(b * _S, _S)],
            out_hbm.at[pl.ds(base + r * _S, _S)], sout.at[b])

    in_copy(0)
    in_copy(1)

    def row_body(r, _carry):
        b = r % _NBUF
        pltpu.make_async_copy(
            x_hbm.at[pl.ds(base + r * _S, _S)],
            xball.at[pl.ds(b * _S, _S)], sin.at[b]).wait()
        xb = xball.at[pl.ds(b * _S, _S)]
        tf = _row_select(xb, cv)

        @plsc.parallel_loop(0, _NVEC // 4, unroll=2)
        def _e(j):
            for u in range(4):
                x16 = xb[pl.ds((j * 4 + u) * _L, _L)]
                xb[pl.ds((j * 4 + u) * _L, _L)] = jnp.where(
                    x16 >= tf, jnp.float32(1.0), jnp.float32(0.0))

        out_copy(r)

        @pl.when(jnp.logical_and(r + _NBUF - 1 < _RPW, r >= 1))
        def _wait_prev_out():
            b2 = (r + _NBUF - 1) % _NBUF
            pltpu.make_async_copy(
                xball.at[pl.ds(b2 * _S, _S)],
                out_hbm.at[pl.ds(0, _S)], sout.at[b2]).wait()

        @pl.when(r + _NBUF - 1 < _RPW)
        def _start_next_in():
            in_copy(r + _NBUF - 1)

        return _carry

    lax.fori_loop(0, _RPW, row_body, 0)

    for r in range(_RPW - _NBUF, _RPW):
        b = r % _NBUF
        pltpu.make_async_copy(
            xball.at[pl.ds(b * _S, _S)],
            out_hbm.at[pl.ds(base + r * _S, _S)], sout.at[b]).wait()


_mesh = plsc.VectorSubcoreMesh(core_axis_name="c", subcore_axis_name="s")

_sc_call = functools.partial(
    pl.kernel,
    out_type=jax.ShapeDtypeStruct((_B * _S,), jnp.float32),
    mesh=_mesh,
    scratch_types=[
        pltpu.VMEM((_NBUF * _S,), jnp.float32),
        pltpu.VMEM((_CAP,), jnp.float32),
        pltpu.SemaphoreType.DMA((_NBUF,)),
        pltpu.SemaphoreType.DMA((_NBUF,)),
    ],
    compiler_params=pltpu.CompilerParams(needs_layout_passes=False),
)(_sc_body)


@jax.jit
def kernel(logits):
    x = logits.reshape(_B * _S)
    return _sc_call(x).reshape(_B, _S, 1)
